# baseline (speedup 1.0000x reference)
import functools

import jax
import jax.numpy as jnp
from jax import lax
from jax.experimental import pallas as pl
from jax.experimental.pallas import tpu as pltpu
from jax.experimental.pallas import tpu_sc as plsc

_NUM_ROWS = 100000
_BATCH = 16384
_D = 2
_N = _BATCH * _D

_info = plsc.get_sparse_core_info()
_NC, _NS = _info.num_cores, _info.num_subcores
_NL = _info.num_lanes
_NW = _NC * _NS
_NCHUNK = _NW // 2
_C_PER_W = _BATCH // _NCHUNK
_GROUPS = _C_PER_W // _NL

_mesh = plsc.VectorSubcoreMesh(core_axis_name="c", subcore_axis_name="s")


@functools.partial(
    pl.kernel,
    mesh=_mesh,
    out_type=jax.ShapeDtypeStruct((_N,), jnp.float32),
    scratch_types=[
        pltpu.VMEM((_C_PER_W,), jnp.int32),
        pltpu.VMEM((_C_PER_W,), jnp.int32),
        pltpu.VMEM((_C_PER_W,), jnp.float32),
        pltpu.SemaphoreType.DMA,
    ],
)
def _gather_col(x_hbm, table_hbm, out_hbm, off_v, opos_v, vals_v, sem):
    wid = lax.axis_index("s") * _NC + lax.axis_index("c")
    chunk = wid >> 1
    parity = wid & 1
    pltpu.sync_copy(x_hbm.at[pl.ds(chunk * _C_PER_W, _C_PER_W)], off_v)
    lane = lax.iota(jnp.int32, _NL)
    obase = chunk * (_C_PER_W * _D) + parity
    for g in range(_GROUPS):
        s = pl.ds(g * _NL, _NL)
        off_v[s] = off_v[s] * _D + parity
        opos_v[s] = (lane + g * _NL) * _D + obase
    pltpu.async_copy(table_hbm.at[off_v], vals_v, sem).wait()
    pltpu.async_copy(vals_v, out_hbm.at[opos_v], sem).wait()


def kernel(x, id_to_gps):
    out = _gather_col(x.astype(jnp.int32), id_to_gps.reshape(-1))
    return out.reshape(_BATCH, _D)


# speedup vs baseline: 2.0580x; 2.0580x over previous
import functools

import jax
import jax.numpy as jnp
from jax import lax
from jax.experimental import pallas as pl
from jax.experimental.pallas import tpu as pltpu
from jax.experimental.pallas import tpu_sc as plsc

_NUM_ROWS = 100000
_BATCH = 16384
_D = 2
_N = _BATCH * _D

_info = plsc.get_sparse_core_info()
_NC, _NS = _info.num_cores, _info.num_subcores
_NL = _info.num_lanes
_C_PER_W = 1024
_GROUPS = _C_PER_W // _NL
_SC_OUT = _N // _NC
_OUT_PER_TILE = _SC_OUT // _NS

_mesh = plsc.VectorSubcoreMesh(core_axis_name="c", subcore_axis_name="s")


@functools.partial(
    pl.kernel,
    mesh=_mesh,
    out_type=jax.ShapeDtypeStruct((_N,), jnp.float32),
    scratch_types=[
        pltpu.VMEM((_C_PER_W,), jnp.int32),
        pltpu.VMEM((_C_PER_W,), jnp.int32),
        pltpu.VMEM((_C_PER_W,), jnp.float32),
        pltpu.VMEM_SHARED((_SC_OUT,), jnp.float32),
        pltpu.SemaphoreType.DMA,
    ],
)
def _gather_col(x_hbm, table_hbm, out_hbm, off_v, opos_v, vals_v, out_sh, sem):
    cid = lax.axis_index("c")
    sid = lax.axis_index("s")
    pair = sid >> 1
    parity = sid & 1
    chunk = cid * (_NS // 2) + pair
    pltpu.sync_copy(x_hbm.at[pl.ds(chunk * _C_PER_W, _C_PER_W)], off_v)
    lane = lax.iota(jnp.int32, _NL)
    lbase = pair * (_C_PER_W * _D) + parity
    for g in range(_GROUPS):
        s = pl.ds(g * _NL, _NL)
        off_v[s] = off_v[s] * _D + parity
        opos_v[s] = (lane + g * _NL) * _D + lbase
    pltpu.async_copy(table_hbm.at[off_v], vals_v, sem).wait()
    pltpu.sync_copy(vals_v, out_sh.at[opos_v])
    plsc.subcore_barrier()
    pltpu.sync_copy(
        out_sh.at[pl.ds(sid * _OUT_PER_TILE, _OUT_PER_TILE)],
        out_hbm.at[pl.ds(cid * _SC_OUT + sid * _OUT_PER_TILE, _OUT_PER_TILE)],
    )


def kernel(x, id_to_gps):
    out = _gather_col(x.astype(jnp.int32), id_to_gps.reshape(-1))
    return out.reshape(_BATCH, _D)
